# baseline (speedup 1.0000x reference)
import jax
import jax.numpy as jnp
from jax import lax
from jax.experimental import pallas as pl
from jax.experimental.pallas import tpu as pltpu
from jax.experimental.pallas import tpu_sc as plsc

MARGIN = 1.0
K_CONST = 1.1
EPS = 1e-6

B = 512
D = 128
NC = 2
NS = 16
NW = NC * NS
I_PER_W = B // NW
NJC = B // 16


def _hsum(v):
    vals = [v[u] for u in range(16)]
    while len(vals) > 1:
        vals = [vals[i] + vals[i + 1] for i in range(0, len(vals), 2)]
    return vals[0]


def _rsqrt_newton(x):
    xi = lax.bitcast_convert_type(x, jnp.int32)
    yi = jnp.int32(0x5F3759DF) - lax.shift_right_logical(xi, 1)
    y = lax.bitcast_convert_type(yi, jnp.float32)
    for _ in range(3):
        y = y * (1.5 - 0.5 * x * y * y)
    return y


def _pairs_body(et_hbm, e_hbm, t_hbm, out_hbm,
                et_v, es_v, tg_v, a_v, po_v):
    core = lax.axis_index("c")
    sid = lax.axis_index("s")
    w = core * NS + sid

    lane = lax.iota(jnp.int32, 16)
    lo = 8 * w
    hi = 504 - 8 * w

    pltpu.sync_copy(et_hbm, et_v)
    pltpu.sync_copy(t_hbm, tg_v)
    pltpu.sync_copy(e_hbm.at[pl.ds(lo, 8)], es_v.at[pl.ds(0, 8)])
    pltpu.sync_copy(e_hbm.at[pl.ds(hi, 8)], es_v.at[pl.ds(8, 8)])

    def a_body(jc, _):
        def d_body(k, carry):
            nacc, sacc = carry
            for u in range(8):
                v = et_v[k * 8 + u, pl.ds(jc * 16, 16)]
                nacc = nacc + v * v
                sacc = sacc + v
            return nacc, sacc
        nv, sv = lax.fori_loop(0, D // 8, d_body,
                               (jnp.zeros((16,), jnp.float32),
                                jnp.zeros((16,), jnp.float32)))
        a_v[jc, :] = nv - (2.0 * EPS) * sv
        return 0
    lax.fori_loop(0, NJC, a_body, 0)

    t_lo = tg_v[pl.ds(lo, 16)]
    t_hi = tg_v[pl.ds(hi - 8, 16)]

    pos_acc = jnp.zeros((16,), jnp.float32)
    neg_acc = jnp.zeros((16,), jnp.float32)

    for ii in range(I_PER_W):
        if ii < 8:
            i_g = lo + ii
            t_i = t_lo[ii]
        else:
            i_g = hi + (ii - 8)
            t_i = t_hi[ii]
        nb = jnp.zeros((16,), jnp.float32)
        sb = jnp.zeros((16,), jnp.float32)
        for k in range(D // 16):
            v = es_v[ii, pl.ds(k * 16, 16)]
            nb = nb + v * v
            sb = sb + v
        b_i = _hsum(nb) + (2.0 * EPS) * _hsum(sb) + (D * EPS * EPS)
        jc0 = lax.shift_right_logical(i_g + 1, 4)

        def jc_body(jc, carry, ii=ii, i_g=i_g, t_i=t_i, b_i=b_i):
            p_acc, n_acc = carry

            def d_body(k, acc, ii=ii, jc=jc):
                ec = es_v[ii, pl.ds(k * 16, 16)]
                for u in range(16):
                    acc = acc + ec[u] * et_v[k * 16 + u, pl.ds(jc * 16, 16)]
                return acc
            g = lax.fori_loop(0, D // 16, d_body,
                              jnp.zeros((16,), jnp.float32))

            d2 = b_i + a_v[jc, :] - 2.0 * g
            d2 = jnp.maximum(d2, 0.0)
            d2s = jnp.maximum(d2, 1e-12)
            dist = d2s * _rsqrt_newton(d2s)

            jvec = jc * 16 * jnp.ones((16,), jnp.int32) + lane
            valid_f = jnp.where(jvec > i_g, 1.0, 0.0)
            same_f = jnp.where(tg_v[pl.ds(jc * 16, 16)] == t_i, 1.0, 0.0)

            pos_t = jnp.maximum(d2 - MARGIN, 0.0)
            neg_h = jnp.maximum(MARGIN * K_CONST - dist, 0.0)
            neg_t = neg_h * neg_h

            p_acc = p_acc + valid_f * same_f * pos_t
            n_acc = n_acc + valid_f * (1.0 - same_f) * neg_t
            return p_acc, n_acc

        pos_acc, neg_acc = lax.fori_loop(jc0, NJC, jc_body,
                                         (pos_acc, neg_acc))

    po_v[0, :] = pos_acc
    po_v[1, :] = neg_acc
    pltpu.sync_copy(po_v, out_hbm.at[w])


def _reduce_body(part_hbm, out_hbm, pa_v, po_v):
    core = lax.axis_index("c")
    sid = lax.axis_index("s")
    lane = lax.iota(jnp.int32, 16)

    @pl.when((core == 0) & (sid == 0))
    def _():
        pltpu.sync_copy(part_hbm, pa_v)
        pt = jnp.zeros((16,), jnp.float32)
        nt = jnp.zeros((16,), jnp.float32)
        for k in range(NW):
            pt = pt + pa_v[k, 0, :]
            nt = nt + pa_v[k, 1, :]
        pos_s = _hsum(pt)
        neg_s = _hsum(nt)
        po_v[0, :] = jnp.where(lane == 0, pos_s, 0.0)
        po_v[1, :] = jnp.where(lane == 0, neg_s, 0.0)
        pltpu.sync_copy(po_v, out_hbm)


@jax.jit
def _sc_loss(et, e, t):
    mesh = plsc.VectorSubcoreMesh(core_axis_name="c", subcore_axis_name="s",
                                  num_cores=NC, num_subcores=NS)
    parts = pl.kernel(
        _pairs_body,
        out_type=jax.ShapeDtypeStruct((NW, 2, 16), jnp.float32),
        mesh=mesh,
        scratch_types=[
            pltpu.VMEM((D, B), jnp.float32),
            pltpu.VMEM((I_PER_W, D), jnp.float32),
            pltpu.VMEM((B,), jnp.int32),
            pltpu.VMEM((NJC, 16), jnp.float32),
            pltpu.VMEM((2, 16), jnp.float32),
        ],
    )(et, e, t)
    mesh2 = plsc.VectorSubcoreMesh(core_axis_name="c", subcore_axis_name="s",
                                   num_cores=NC, num_subcores=NS)
    return pl.kernel(
        _reduce_body,
        out_type=jax.ShapeDtypeStruct((2, 16), jnp.float32),
        mesh=mesh2,
        scratch_types=[
            pltpu.VMEM((NW, 2, 16), jnp.float32),
            pltpu.VMEM((2, 16), jnp.float32),
        ],
    )(parts)


def kernel(embeddings, target):
    e = embeddings.astype(jnp.float32)
    et = e.T.copy()
    t = target.astype(jnp.int32)
    out = _sc_loss(et, e, t)
    return (out[0, 0], out[1, 0])


# speedup vs baseline: 19.2421x; 19.2421x over previous
import jax
import jax.numpy as jnp
from jax.experimental import pallas as pl

MARGIN = 1.0
K_CONST = 1.1
EPS = 1e-6


def _loss_kernel(e_ref, t_ref, pos_ref, neg_ref):
    e = e_ref[...]
    t = t_ref[...]
    B = e.shape[0]
    D = e.shape[1]

    eb = e.astype(jnp.bfloat16)
    g = jax.lax.dot_general(
        eb, eb,
        dimension_numbers=(((1,), (1,)), ((), ())),
        preferred_element_type=jnp.float32,
        precision=jax.lax.Precision.DEFAULT,
    )
    n = jnp.sum(e * e, axis=1, keepdims=True)
    s = jnp.sum(e, axis=1, keepdims=True)

    d2 = n + n.T - 2.0 * g + (2.0 * EPS) * (s - s.T) + (D * EPS * EPS)
    d2 = jnp.maximum(d2, 0.0)
    dist = jnp.sqrt(d2)

    pos_terms = jnp.maximum(d2 - MARGIN, 0.0)
    neg_hinge = jnp.maximum(MARGIN * K_CONST - dist, 0.0)
    neg_terms = neg_hinge * neg_hinge

    row = jax.lax.broadcasted_iota(jnp.int32, (B, B), 0)
    col = jax.lax.broadcasted_iota(jnp.int32, (B, B), 1)
    upper = col > row
    same = t == t.T

    pos_ref[...] = jnp.sum(
        jnp.where(upper & same, pos_terms, 0.0)).reshape(1, 1)
    neg_ref[...] = jnp.sum(
        jnp.where(upper & (~same), neg_terms, 0.0)).reshape(1, 1)


def kernel(embeddings, target):
    B = embeddings.shape[0]
    t2d = target.astype(jnp.int32).reshape(B, 1)
    pos, neg = pl.pallas_call(
        _loss_kernel,
        out_shape=(
            jax.ShapeDtypeStruct((1, 1), jnp.float32),
            jax.ShapeDtypeStruct((1, 1), jnp.float32),
        ),
    )(embeddings.astype(jnp.float32), t2d)
    return (pos[0, 0], neg[0, 0])
